# baseline (speedup 1.0000x reference)
import functools

import jax
import jax.numpy as jnp
from jax.experimental import pallas as pl
from jax.experimental.pallas import tpu as pltpu

EPS = 1e-5


def _fused_body(x_ref, b1_ref, w1_ref, b2_ref, w2_ref, m_ref, o_ref,
                y2buf, xsem, ysem, *, H, W, NGRID):
    Cin = x_ref.shape[1]
    HW = H * W
    Cout = w2_ref.shape[0] // 9

    i = pl.program_id(0)
    slot = jax.lax.rem(i, 2)

    cpx = pltpu.make_async_copy(x_ref.at[0], o_ref.at[i, pl.ds(0, Cin), :], xsem)
    cpx.start()

    @pl.when(i >= 2)
    def _():
        pltpu.make_async_copy(
            y2buf.at[slot], o_ref.at[i - 2, pl.ds(Cin, Cout), :],
            ysem.at[slot]).wait()

    x = x_ref[0]
    z = jnp.maximum(x + b1_ref[...], 0.0).astype(jnp.bfloat16)

    y1 = jnp.dot(w1_ref[...], z, preferred_element_type=jnp.float32)
    y1 = jnp.maximum(y1 + b2_ref[...], 0.0).astype(jnp.bfloat16)

    acc = None
    for g in range(3):
        p = jnp.dot(w2_ref[3 * Cout * g:3 * Cout * (g + 1), :], y1,
                    preferred_element_type=jnp.float32)
        for j in range(3):
            t = 3 * g + j
            ky, kx = divmod(t, 3)
            off = (ky - 1) * W + (kx - 1)
            pt = p[Cout * j:Cout * (j + 1), :]
            if off:
                pt = pltpu.roll(pt, (-off) % HW, axis=1)
            term = pt * m_ref[t:t + 1, :]
            acc = term if acc is None else acc + term
    y2buf[slot] = acc

    cpy = pltpu.make_async_copy(
        y2buf.at[slot], o_ref.at[i, pl.ds(Cin, Cout), :], ysem.at[slot])
    cpy.start()

    @pl.when(i == NGRID - 1)
    def _():
        pltpu.make_async_copy(
            y2buf.at[1 - slot], o_ref.at[i - 1, pl.ds(Cin, Cout), :],
            ysem.at[1 - slot]).wait()
        cpy.wait()

    cpx.wait()


def kernel(x, w1, w2, g1, be1, m1, v1, g2, be2, m2, v2):
    N, Cin, H, W = x.shape
    Cb, Cout = w1.shape[0], w2.shape[0]
    HW = H * W
    f32 = jnp.float32

    s1 = g1 / jnp.sqrt(v1 + EPS)
    b1 = be1 - m1 * s1
    s2 = g2 / jnp.sqrt(v2 + EPS)
    b2 = be2 - m2 * s2

    w1_mat = (w1[:, :, 0, 0].astype(f32) * s2[:, None] * s1[None, :]
              ).astype(jnp.bfloat16)
    b1c = (b1 / s1).astype(f32)[:, None]
    w2_all = (jnp.transpose(w2.astype(f32), (2, 3, 0, 1))
              .reshape(9 * Cout, Cb).astype(jnp.bfloat16))

    b2c = b2.astype(f32)[:, None]

    q = jnp.arange(HW, dtype=jnp.int32)
    hh = q // W
    ww = q % W
    masks = []
    for t in range(9):
        ky, kx = divmod(t, 3)
        dy, dx = ky - 1, kx - 1
        m = ((hh + dy >= 0) & (hh + dy < H) & (ww + dx >= 0) & (ww + dx < W))
        masks.append(m.astype(f32))
    m_all = jnp.stack(masks, axis=0)

    x3 = x.reshape(N, Cin, HW)

    bytes_in = Cin * HW * 4
    bytes_out = (Cin + Cout) * HW * 4
    bytes_w = Cb * Cin * 2 + 9 * Cout * Cb * 2 + 9 * HW * 4 + (Cin + Cb) * 4
    flops = 2 * N * (HW * Cin * Cb + 9 * HW * Cb * Cout)

    out = pl.pallas_call(
        functools.partial(_fused_body, H=H, W=W, NGRID=N),
        out_shape=jax.ShapeDtypeStruct((N, Cin + Cout, HW), f32),
        grid=(N,),
        in_specs=[
            pl.BlockSpec((1, Cin, HW), lambda i: (i, 0, 0)),
            pl.BlockSpec((Cin, 1), lambda i: (0, 0)),
            pl.BlockSpec((Cb, Cin), lambda i: (0, 0)),
            pl.BlockSpec((Cb, 1), lambda i: (0, 0)),
            pl.BlockSpec((9 * Cout, Cb), lambda i: (0, 0)),
            pl.BlockSpec((9, HW), lambda i: (0, 0)),
        ],
        out_specs=pl.BlockSpec(memory_space=pltpu.MemorySpace.HBM),
        scratch_shapes=[
            pltpu.VMEM((2, Cout, HW), f32),
            pltpu.SemaphoreType.DMA,
            pltpu.SemaphoreType.DMA((2,)),
        ],
        compiler_params=pltpu.CompilerParams(
            dimension_semantics=("arbitrary",),
            vmem_limit_bytes=int(48 * 2**20),
        ),
        cost_estimate=pl.CostEstimate(
            flops=flops, transcendentals=0,
            bytes_accessed=N * (bytes_in + bytes_out) + bytes_w),
    )(x3, b1c, w1_mat, b2c, w2_all, m_all)

    return out.reshape(N, Cin + Cout, H, W)
